# baseline (speedup 1.0000x reference)
import functools
import math

import jax
import jax.numpy as jnp
from jax import lax
from jax.experimental import pallas as pl
from jax.experimental.pallas import tpu as pltpu
from jax.experimental.pallas import tpu_sc as plsc

NUM_ENT = 1000000
D = 33
B = 16384
NC = 2
NS = 16
NW = NC * NS
BPW = B // NW
CHUNK = 256
L = 16
CLAMP = 1.0 + 1e-6

_mesh = plsc.VectorSubcoreMesh(
    core_axis_name="c", subcore_axis_name="s", num_cores=NC, num_subcores=NS
)


@functools.partial(
    pl.kernel,
    out_type=jax.ShapeDtypeStruct((B,), jnp.float32),
    mesh=_mesh,
    scratch_types=[
        pltpu.VMEM((BPW,), jnp.int32),
        pltpu.VMEM((BPW,), jnp.int32),
        pltpu.VMEM((CHUNK, D), jnp.float32),
        pltpu.VMEM((CHUNK, D), jnp.float32),
        pltpu.VMEM((BPW,), jnp.float32),
        pltpu.SemaphoreType.DMA,
    ],
    compiler_params=pltpu.CompilerParams(
        needs_layout_passes=False, use_tc_tiling_on_sc=True
    ),
)
def _sc_neg_inner(table, head, tail, out, hidx, tidx, urows, vrows, res, sem):
    wid = lax.axis_index("s") * NC + lax.axis_index("c")
    base = wid * BPW

    pltpu.sync_copy(head.at[pl.ds(base, BPW)], hidx)
    pltpu.sync_copy(tail.at[pl.ds(base, BPW)], tidx)

    lanes = lax.iota(jnp.int32, L)

    def half(h, carry):
        def fire(g, carry):
            hv = hidx[pl.ds(h * CHUNK + g * L, L)]
            tv = tidx[pl.ds(h * CHUNK + g * L, L)]
            for j in range(L):
                n = g * L + j
                pltpu.async_copy(
                    table.at[pl.ds(hv[j], 1)], urows.at[pl.ds(n, 1)], sem
                )
                pltpu.async_copy(
                    table.at[pl.ds(tv[j], 1)], vrows.at[pl.ds(n, 1)], sem
                )
            return carry

        lax.fori_loop(0, CHUNK // L, fire, 0)

        pltpu.make_async_copy(table.at[pl.ds(0, CHUNK)], urows, sem).wait()
        pltpu.make_async_copy(table.at[pl.ds(0, CHUNK)], vrows, sem).wait()

        def body(g, carry):
            row = jnp.full((L,), g * L, jnp.int32) + lanes
            col0 = jnp.zeros((L,), jnp.int32)
            u0 = plsc.load_gather(urows, [row, col0])
            v0 = plsc.load_gather(vrows, [row, col0])
            acc = u0 * v0
            for j in range(1, D):
                cj = jnp.full((L,), j, jnp.int32)
                acc = acc - plsc.load_gather(
                    urows, [row, cj]
                ) * plsc.load_gather(vrows, [row, cj])
            res[pl.ds(h * CHUNK + g * L, L)] = acc
            return carry

        lax.fori_loop(0, CHUNK // L, body, 0)
        return carry

    lax.fori_loop(0, BPW // CHUNK, half, 0)
    pltpu.sync_copy(res, out.at[pl.ds(base, BPW)])


def _acosh_body(x_ref, o_ref):
    x = jnp.maximum(x_ref[...], CLAMP)
    o_ref[...] = jnp.log(x + jnp.sqrt((x - 1.0) * (x + 1.0)))


_acosh_tc = pl.pallas_call(
    _acosh_body,
    out_shape=jax.ShapeDtypeStruct((128, 128), jnp.float32),
)


def _transpose_body(x_ref, o_ref):
    o_ref[...] = x_ref[...].T


_TBLK = 512
_transpose_tc = pl.pallas_call(
    _transpose_body,
    grid=(pl.cdiv(NUM_ENT, _TBLK),),
    in_specs=[pl.BlockSpec((D, _TBLK), lambda i: (0, i))],
    out_specs=pl.BlockSpec((_TBLK, D), lambda i: (i, 0)),
    out_shape=jax.ShapeDtypeStruct((NUM_ENT, D), jnp.float32),
)


def kernel(entity_embeddings, head_idx, tail_idx):
    h = head_idx.astype(jnp.int32)
    t = tail_idx.astype(jnp.int32)
    table_rm = _transpose_tc(entity_embeddings.T)
    neg_inner = _sc_neg_inner(table_rm, h, t)
    dist = _acosh_tc(neg_inner.reshape(128, 128))
    return dist.reshape(B)


# speedup vs baseline: 4.2527x; 4.2527x over previous
import functools
import math

import jax
import jax.numpy as jnp
from jax import lax
from jax.experimental import pallas as pl
from jax.experimental.pallas import tpu as pltpu
from jax.experimental.pallas import tpu_sc as plsc

NUM_ENT = 1000000
D = 33
B = 16384
NC = 2
NS = 16
NW = NC * NS
BPW = B // NW
CHUNK = 256
L = 16
CLAMP = 1.0 + 1e-6

_mesh = plsc.VectorSubcoreMesh(
    core_axis_name="c", subcore_axis_name="s", num_cores=NC, num_subcores=NS
)


@functools.partial(
    pl.kernel,
    out_type=jax.ShapeDtypeStruct((B,), jnp.float32),
    mesh=_mesh,
    scratch_types=[
        pltpu.VMEM((BPW,), jnp.int32),
        pltpu.VMEM((BPW,), jnp.int32),
        pltpu.VMEM((CHUNK, D), jnp.float32),
        pltpu.VMEM((CHUNK, D), jnp.float32),
        pltpu.VMEM((BPW,), jnp.float32),
        pltpu.SemaphoreType.DMA,
    ],
    compiler_params=pltpu.CompilerParams(
        needs_layout_passes=False, use_tc_tiling_on_sc=True
    ),
)
def _sc_neg_inner(table, head, tail, out, hidx, tidx, urows, vrows, res, sem):
    wid = lax.axis_index("s") * NC + lax.axis_index("c")
    base = wid * BPW

    pltpu.sync_copy(head.at[pl.ds(base, BPW)], hidx)
    pltpu.sync_copy(tail.at[pl.ds(base, BPW)], tidx)

    lanes = lax.iota(jnp.int32, L)

    def half(h, carry):
        def fire(g, carry):
            hv = hidx[pl.ds(h * CHUNK + g * L, L)]
            tv = tidx[pl.ds(h * CHUNK + g * L, L)]
            for j in range(L):
                n = g * L + j
                pltpu.async_copy(
                    table.at[pl.ds(hv[j], 1)], urows.at[pl.ds(n, 1)], sem
                )
                pltpu.async_copy(
                    table.at[pl.ds(tv[j], 1)], vrows.at[pl.ds(n, 1)], sem
                )
            return carry

        lax.fori_loop(0, CHUNK // L, fire, 0)

        pltpu.make_async_copy(table.at[pl.ds(0, CHUNK)], urows, sem).wait()
        pltpu.make_async_copy(table.at[pl.ds(0, CHUNK)], vrows, sem).wait()

        def body(g, carry):
            row = jnp.full((L,), g * L, jnp.int32) + lanes
            col0 = jnp.zeros((L,), jnp.int32)
            u0 = plsc.load_gather(urows, [row, col0])
            v0 = plsc.load_gather(vrows, [row, col0])
            acc = u0 * v0
            for j in range(1, D):
                cj = jnp.full((L,), j, jnp.int32)
                acc = acc - plsc.load_gather(
                    urows, [row, cj]
                ) * plsc.load_gather(vrows, [row, cj])
            res[pl.ds(h * CHUNK + g * L, L)] = acc
            return carry

        lax.fori_loop(0, CHUNK // L, body, 0)
        return carry

    lax.fori_loop(0, BPW // CHUNK, half, 0)
    pltpu.sync_copy(res, out.at[pl.ds(base, BPW)])


def _acosh_body(x_ref, o_ref):
    x = jnp.maximum(x_ref[...], CLAMP)
    o_ref[...] = jnp.log(x + jnp.sqrt((x - 1.0) * (x + 1.0)))


_acosh_tc = pl.pallas_call(
    _acosh_body,
    out_shape=jax.ShapeDtypeStruct((128, 128), jnp.float32),
)


def _transpose_body(x_ref, o_ref):
    o_ref[...] = x_ref[...].T


_TBLK = 8192
_transpose_tc = pl.pallas_call(
    _transpose_body,
    grid=(pl.cdiv(NUM_ENT, _TBLK),),
    in_specs=[pl.BlockSpec((D, _TBLK), lambda i: (0, i))],
    out_specs=pl.BlockSpec((_TBLK, D), lambda i: (i, 0)),
    out_shape=jax.ShapeDtypeStruct((NUM_ENT, D), jnp.float32),
)


def kernel(entity_embeddings, head_idx, tail_idx):
    h = head_idx.astype(jnp.int32)
    t = tail_idx.astype(jnp.int32)
    table_rm = _transpose_tc(entity_embeddings.T)
    neg_inner = _sc_neg_inner(table_rm, h, t)
    dist = _acosh_tc(neg_inner.reshape(128, 128))
    return dist.reshape(B)


# speedup vs baseline: 4.8556x; 1.1418x over previous
import functools
import math

import jax
import jax.numpy as jnp
from jax import lax
from jax.experimental import pallas as pl
from jax.experimental.pallas import tpu as pltpu
from jax.experimental.pallas import tpu_sc as plsc

NUM_ENT = 1000000
D = 33
B = 16384
NC = 2
NS = 16
NW = NC * NS
BPW = B // NW
CHUNK = 256
L = 16
CLAMP = 1.0 + 1e-6

_mesh = plsc.VectorSubcoreMesh(
    core_axis_name="c", subcore_axis_name="s", num_cores=NC, num_subcores=NS
)


@functools.partial(
    pl.kernel,
    out_type=jax.ShapeDtypeStruct((B,), jnp.float32),
    mesh=_mesh,
    scratch_types=[
        pltpu.VMEM((BPW,), jnp.int32),
        pltpu.VMEM((BPW,), jnp.int32),
        pltpu.VMEM((CHUNK, D), jnp.float32),
        pltpu.VMEM((CHUNK, D), jnp.float32),
        pltpu.VMEM((BPW,), jnp.float32),
        pltpu.SemaphoreType.DMA,
    ],
    compiler_params=pltpu.CompilerParams(
        needs_layout_passes=False, use_tc_tiling_on_sc=True
    ),
)
def _sc_neg_inner(table, head, tail, out, hidx, tidx, urows, vrows, res, sem):
    wid = lax.axis_index("s") * NC + lax.axis_index("c")
    base = wid * BPW

    pltpu.sync_copy(head.at[pl.ds(base, BPW)], hidx)
    pltpu.sync_copy(tail.at[pl.ds(base, BPW)], tidx)

    lanes = lax.iota(jnp.int32, L)

    def half(h, carry):
        def fire(g, carry):
            hv = hidx[pl.ds(h * CHUNK + g * L, L)]
            tv = tidx[pl.ds(h * CHUNK + g * L, L)]
            for j in range(L):
                n = g * L + j
                pltpu.async_copy(
                    table.at[pl.ds(hv[j], 1)], urows.at[pl.ds(n, 1)], sem
                )
                pltpu.async_copy(
                    table.at[pl.ds(tv[j], 1)], vrows.at[pl.ds(n, 1)], sem
                )
            return carry

        lax.fori_loop(0, CHUNK // L, fire, 0)

        pltpu.make_async_copy(table.at[pl.ds(0, CHUNK)], urows, sem).wait()
        pltpu.make_async_copy(table.at[pl.ds(0, CHUNK)], vrows, sem).wait()

        def body(g, carry):
            row = jnp.full((L,), g * L, jnp.int32) + lanes
            col0 = jnp.zeros((L,), jnp.int32)
            u0 = plsc.load_gather(urows, [row, col0])
            v0 = plsc.load_gather(vrows, [row, col0])
            acc = u0 * v0
            for j in range(1, D):
                cj = jnp.full((L,), j, jnp.int32)
                acc = acc - plsc.load_gather(
                    urows, [row, cj]
                ) * plsc.load_gather(vrows, [row, cj])
            res[pl.ds(h * CHUNK + g * L, L)] = acc
            return carry

        lax.fori_loop(0, CHUNK // L, body, 0)
        return carry

    lax.fori_loop(0, BPW // CHUNK, half, 0)
    pltpu.sync_copy(res, out.at[pl.ds(base, BPW)])


def _acosh_body(x_ref, o_ref):
    x = jnp.maximum(x_ref[...], CLAMP)
    o_ref[...] = jnp.log(x + jnp.sqrt((x - 1.0) * (x + 1.0)))


_acosh_tc = pl.pallas_call(
    _acosh_body,
    out_shape=jax.ShapeDtypeStruct((128, 128), jnp.float32),
)


def _transpose_body(x_ref, o_ref):
    o_ref[...] = x_ref[...].T


_TBLK = 32768
_transpose_tc = pl.pallas_call(
    _transpose_body,
    grid=(pl.cdiv(NUM_ENT, _TBLK),),
    in_specs=[pl.BlockSpec((D, _TBLK), lambda i: (0, i))],
    out_specs=pl.BlockSpec((_TBLK, D), lambda i: (i, 0)),
    out_shape=jax.ShapeDtypeStruct((NUM_ENT, D), jnp.float32),
)


def kernel(entity_embeddings, head_idx, tail_idx):
    h = head_idx.astype(jnp.int32)
    t = tail_idx.astype(jnp.int32)
    table_rm = _transpose_tc(entity_embeddings.T)
    neg_inner = _sc_neg_inner(table_rm, h, t)
    dist = _acosh_tc(neg_inner.reshape(128, 128))
    return dist.reshape(B)
